# Initial kernel scaffold; baseline (speedup 1.0000x reference)
import functools

import jax
import jax.numpy as jnp
from jax import lax
from jax.experimental import pallas as pl
from jax.experimental.pallas import tpu as pltpu
from jax.experimental.pallas import tpu_sc as plsc

_NC = 2
_NS = 16
_NW = _NC * _NS


def _make_gather(n, v, d, chunk):
    assert n % (_NW * chunk) == 0
    b_per_w = n // _NW
    n_chunks = b_per_w // chunk
    mesh = plsc.VectorSubcoreMesh(core_axis_name="c", subcore_axis_name="s")

    @functools.partial(
        pl.kernel,
        mesh=mesh,
        out_type=jax.ShapeDtypeStruct((n, d), jnp.float32),
        scratch_types=[
            pltpu.VMEM((chunk,), jnp.int32),
            pltpu.VMEM((chunk, d), jnp.float32),
            pltpu.SemaphoreType.DMA,
        ],
    )
    def gather_kernel(idx_hbm, table_hbm, out_hbm, idx_v, rows_v, sem):
        wid = lax.axis_index("s") * _NC + lax.axis_index("c")
        base = wid * b_per_w

        @pl.loop(0, n_chunks)
        def _(i):
            off = base + i * chunk
            pltpu.sync_copy(idx_hbm.at[pl.ds(off, chunk)], idx_v)
            pltpu.async_copy(table_hbm.at[idx_v], rows_v, sem).wait()
            pltpu.sync_copy(rows_v, out_hbm.at[pl.ds(off, chunk)])

    return gather_kernel


@jax.jit
def kernel(x, table):
    batch, hist = x.shape
    vocab, dim = table.shape
    n = batch * hist
    flat_idx = x.reshape(n)
    out = _make_gather(n, vocab, dim, 1024)(flat_idx, table)
    return out.reshape(batch, hist, dim)


# speedup vs baseline: 1.8453x; 1.8453x over previous
import functools

import jax
import jax.numpy as jnp
from jax import lax
from jax.experimental import pallas as pl
from jax.experimental.pallas import tpu as pltpu
from jax.experimental.pallas import tpu_sc as plsc

_NC = 2
_NS = 16
_NW = _NC * _NS


def _make_gather(n, v, d, chunk):
    assert n % (_NW * chunk) == 0
    b_per_w = n // _NW
    n_chunks = b_per_w // chunk
    mesh = plsc.VectorSubcoreMesh(core_axis_name="c", subcore_axis_name="s")

    @functools.partial(
        pl.kernel,
        mesh=mesh,
        out_type=jax.ShapeDtypeStruct((n, d), jnp.float32),
        scratch_types=[
            pltpu.VMEM((chunk,), jnp.int32),
            pltpu.VMEM((chunk, d), jnp.float32),
            pltpu.SemaphoreType.DMA,
        ],
        compiler_params=pltpu.CompilerParams(use_tc_tiling_on_sc=False),
    )
    def gather_kernel(idx_hbm, table_hbm, out_hbm, idx_v, rows_v, sem):
        wid = lax.axis_index("s") * _NC + lax.axis_index("c")
        base = wid * b_per_w

        @pl.loop(0, n_chunks)
        def _(i):
            off = base + i * chunk
            pltpu.sync_copy(idx_hbm.at[pl.ds(off, chunk)], idx_v)
            pltpu.async_copy(table_hbm.at[idx_v], rows_v, sem).wait()
            pltpu.sync_copy(rows_v, out_hbm.at[pl.ds(off, chunk)])

    return gather_kernel


@jax.jit
def kernel(x, table):
    batch, hist = x.shape
    vocab, dim = table.shape
    n = batch * hist
    flat_idx = x.reshape(n)
    out = _make_gather(n, vocab, dim, 1024)(flat_idx, table)
    return out.reshape(batch, hist, dim)


# speedup vs baseline: 1.8749x; 1.0161x over previous
import functools

import jax
import jax.numpy as jnp
from jax import lax
from jax.experimental import pallas as pl
from jax.experimental.pallas import tpu as pltpu
from jax.experimental.pallas import tpu_sc as plsc

_NC = 2
_NS = 16
_NW = _NC * _NS


def _make_gather(n, v, d, chunk):
    assert n % (_NW * chunk) == 0
    b_per_w = n // _NW
    n_chunks = b_per_w // chunk
    assert n_chunks >= 4 and n_chunks % 2 == 0
    mesh = plsc.VectorSubcoreMesh(core_axis_name="c", subcore_axis_name="s")

    @functools.partial(
        pl.kernel,
        mesh=mesh,
        out_type=jax.ShapeDtypeStruct((n, d), jnp.float32),
        scratch_types=[
            pltpu.VMEM((chunk,), jnp.int32),
            pltpu.VMEM((chunk,), jnp.int32),
            pltpu.VMEM((chunk, d), jnp.float32),
            pltpu.VMEM((chunk, d), jnp.float32),
            pltpu.SemaphoreType.DMA,
            pltpu.SemaphoreType.DMA,
            pltpu.SemaphoreType.DMA,
            pltpu.SemaphoreType.DMA,
        ],
        compiler_params=pltpu.CompilerParams(use_tc_tiling_on_sc=False),
    )
    def gather_kernel(idx_hbm, table_hbm, out_hbm, idx0, idx1, rows0, rows1,
                      g0, g1, o0, o1):
        idx_v = [idx0, idx1]
        rows_v = [rows0, rows1]
        gsem = [g0, g1]
        osem = [o0, o1]
        wid = lax.axis_index("s") * _NC + lax.axis_index("c")
        base = wid * b_per_w

        def load_idx(i, slot):
            pltpu.sync_copy(idx_hbm.at[pl.ds(base + i * chunk, chunk)],
                            idx_v[slot])

        def fire_gather(slot):
            pltpu.async_copy(table_hbm.at[idx_v[slot]], rows_v[slot],
                             gsem[slot])

        def wait_gather(slot):
            pltpu.make_async_copy(table_hbm.at[idx_v[slot]],
                                  rows_v[slot], gsem[slot]).wait()

        def fire_store(i, slot):
            pltpu.async_copy(rows_v[slot],
                             out_hbm.at[pl.ds(base + i * chunk, chunk)],
                             osem[slot])

        def wait_store(i, slot):
            pltpu.make_async_copy(rows_v[slot],
                                  out_hbm.at[pl.ds(base + i * chunk, chunk)],
                                  osem[slot]).wait()

        load_idx(0, 0)
        fire_gather(0)
        load_idx(1, 1)
        fire_gather(1)
        wait_gather(0)
        fire_store(0, 0)

        @pl.loop(0, (n_chunks - 2) // 2)
        def _(ii):
            for b in range(2):
                i = 1 + 2 * ii + b
                s = (1 + b) & 1
                o = 1 - s
                load_idx(i + 1, o)
                wait_store(i - 1, o)
                fire_gather(o)
                wait_gather(s)
                fire_store(i, s)

        wait_gather(1)
        fire_store(n_chunks - 1, 1)
        wait_store(n_chunks - 2, 0)
        wait_store(n_chunks - 1, 1)

    return gather_kernel


@jax.jit
def kernel(x, table):
    batch, hist = x.shape
    vocab, dim = table.shape
    n = batch * hist
    flat_idx = x.reshape(n)
    out = _make_gather(n, vocab, dim, 800)(flat_idx, table)
    return out.reshape(batch, hist, dim)


# speedup vs baseline: 1.8760x; 1.0006x over previous
import functools

import jax
import jax.numpy as jnp
from jax import lax
from jax.experimental import pallas as pl
from jax.experimental.pallas import tpu as pltpu
from jax.experimental.pallas import tpu_sc as plsc

_NC = 2
_NS = 16
_NW = _NC * _NS


def _make_gather(n, v, d, chunk, nslots):
    assert n % (_NW * chunk) == 0
    b_per_w = n // _NW
    n_chunks = b_per_w // chunk
    assert n_chunks % nslots == 0 and n_chunks >= 2 * nslots

    mesh = plsc.VectorSubcoreMesh(core_axis_name="c", subcore_axis_name="s")
    scratch = (
        [pltpu.VMEM((chunk,), jnp.int32) for _ in range(nslots)]
        + [pltpu.VMEM((chunk, d), jnp.float32) for _ in range(nslots)]
        + [pltpu.SemaphoreType.DMA for _ in range(2 * nslots)]
    )

    @functools.partial(
        pl.kernel,
        mesh=mesh,
        out_type=jax.ShapeDtypeStruct((n, d), jnp.float32),
        scratch_types=scratch,
        compiler_params=pltpu.CompilerParams(use_tc_tiling_on_sc=False),
    )
    def gather_kernel(idx_hbm, table_hbm, out_hbm, *refs):
        idx_v = list(refs[0:nslots])
        rows_v = list(refs[nslots:2 * nslots])
        gsem = list(refs[2 * nslots:3 * nslots])
        osem = list(refs[3 * nslots:4 * nslots])
        wid = lax.axis_index("s") * _NC + lax.axis_index("c")
        base = wid * b_per_w

        def load_idx(i, slot):
            pltpu.sync_copy(idx_hbm.at[pl.ds(base + i * chunk, chunk)],
                            idx_v[slot])

        def fire_gather(slot):
            pltpu.async_copy(table_hbm.at[idx_v[slot]], rows_v[slot],
                             gsem[slot])

        def wait_gather(slot):
            pltpu.make_async_copy(table_hbm.at[idx_v[slot]],
                                  rows_v[slot], gsem[slot]).wait()

        def fire_store(i, slot):
            pltpu.async_copy(rows_v[slot],
                             out_hbm.at[pl.ds(base + i * chunk, chunk)],
                             osem[slot])

        def wait_store(i, slot):
            pltpu.make_async_copy(rows_v[slot],
                                  out_hbm.at[pl.ds(base + i * chunk, chunk)],
                                  osem[slot]).wait()

        for j in range(nslots - 1):
            load_idx(j, j)
            fire_gather(j)

        load_idx(nslots - 1, nslots - 1)
        fire_gather(nslots - 1)
        wait_gather(0)
        fire_store(0, 0)

        @pl.loop(0, (n_chunks - nslots) // nslots)
        def _(ii):
            for b in range(nslots):
                i = 1 + nslots * ii + b
                s = (1 + b) % nslots
                p = b % nslots
                load_idx(i + nslots - 1, p)
                wait_store(i - 1, p)
                fire_gather(p)
                wait_gather(s)
                fire_store(i, s)

        for i in range(n_chunks - nslots + 1, n_chunks):
            s = i % nslots
            wait_gather(s)
            fire_store(i, s)

        for i in range(n_chunks - nslots, n_chunks):
            wait_store(i, i % nslots)

    return gather_kernel


@jax.jit
def kernel(x, table):
    batch, hist = x.shape
    vocab, dim = table.shape
    n = batch * hist
    flat_idx = x.reshape(n)
    out = _make_gather(n, vocab, dim, 400, 4)(flat_idx, table)
    return out.reshape(batch, hist, dim)
